# baseline (speedup 1.0000x reference)
import functools

import jax
import jax.numpy as jnp
from jax import lax
from jax.experimental import pallas as pl
from jax.experimental.pallas import tpu as pltpu
from jax.experimental.pallas import tpu_sc as plsc

N_USERS = 1340
N_ITEMS = 733
N_F = 5
BIN = 60
MAXDAY = 4097
B = 16384
HIST = 50
BETA = 0.4
GMEAN = 4.0

NS = 16
L = 16

BPW = B // NS
UPS = 96
NT_P1 = 14
LASTOFF = N_USERS - UPS
USPLIT = 13 * UPS
UPAD = NT_P1 * UPS
NCHUNK = BPW // L

LN2 = 0.6931471805599453
SQRT2 = 1.4142135623730951


def _align8(x):
  return (x + 7) & ~7


_F_SIZES = [8, N_USERS * N_F, N_USERS, N_USERS, N_USERS, N_USERS,
            N_USERS * N_F, N_USERS * N_F, N_ITEMS, N_ITEMS * N_F,
            N_ITEMS * BIN, MAXDAY, MAXDAY, MAXDAY * N_F, N_F, N_F]
_F_OFFS = []
_off = 0
for _sz in _F_SIZES:
  _F_OFFS.append(_off)
  _off = _align8(_off + _sz)
(F_PAD, F_Y, F_BU, F_AL, F_MU, F_BCU, F_WPU, F_AUK, F_BI, F_WPI, F_WBIT,
 F_BTD, F_WCU, F_PKUT, F_OW, F_OB) = _F_OFFS
F_TOTAL = _off
F_YEND = F_BU

I_URI = 0
I_CNT = _align8(I_URI + N_USERS * HIST)
I_IDX = _align8(I_CNT + N_USERS)
I_TOTAL = I_IDX + NS * 5 * BPW


def _ln(x):
  bits = lax.bitcast_convert_type(x, jnp.int32)
  e = lax.shift_right_logical(bits, 23) - 127
  m = lax.bitcast_convert_type(
      jnp.bitwise_or(jnp.bitwise_and(bits, 0x007FFFFF), 0x3F800000),
      jnp.float32)
  big = m > SQRT2
  m = jnp.where(big, m * 0.5, m)
  e = (e + jnp.where(big, 1, 0)).astype(jnp.float32)
  z = (m - 1.0) / (m + 1.0)
  z2 = z * z
  p = z * (2.0 + z2 * (2.0 / 3.0 + z2 * (0.4 + z2 * (2.0 / 7.0 + z2 * (2.0 / 9.0)))))
  return e * LN2 + p


def _body(fblob, iblob, out,
          tab_v, uri_v, cnt_v, idx_v, yimpl_v, stage_v, out_v, yimpl_sh,
          sem_a, sem_b):
  s = lax.axis_index("s")
  base = s * BPW
  iota = lax.iota(jnp.int32, L)

  off_u = jnp.minimum(s * UPS, LASTOFF)
  da = [pltpu.async_copy(iblob.at[pl.ds(I_URI + off_u * HIST, UPS * HIST)],
                         uri_v, sem_a),
        pltpu.async_copy(iblob.at[pl.ds(I_CNT, N_USERS)], cnt_v, sem_a),
        pltpu.async_copy(fblob.at[pl.ds(0, F_YEND)],
                         tab_v.at[pl.ds(0, F_YEND)], sem_a)]
  db = [pltpu.async_copy(fblob.at[pl.ds(F_YEND, F_TOTAL - F_YEND)],
                         tab_v.at[pl.ds(F_YEND, F_TOTAL - F_YEND)], sem_b),
        pltpu.async_copy(iblob.at[pl.ds(I_IDX + s * (5 * BPW), 5 * BPW)],
                         idx_v, sem_b)]
  for d in da:
    d.wait()

  @pl.when(s < NT_P1)
  def _phase1():
    def chunk_step(chunk):
      rows = iota + chunk * L
      rbase = rows * HIST

      def h_step(h, accs):
        hidx = plsc.load_gather(uri_v, [rbase + h])
        ybase = hidx * N_F + F_Y
        return tuple(
            acc + plsc.load_gather(tab_v, [ybase + f])
            for f, acc in enumerate(accs))

      accs = plsc.parallel_loop(
          0, HIST, unroll=2,
          carry=tuple(jnp.zeros((L,), jnp.float32) for _ in range(N_F)))(h_step)
      cntf = plsc.load_gather(cnt_v, [off_u + rows]).astype(jnp.float32)
      ru = jnp.exp(-0.5 * _ln(cntf))
      sbase = rows * N_F
      for f in range(N_F):
        plsc.store_scatter(stage_v, [sbase + f], accs[f] * ru)

    plsc.parallel_loop(0, UPS // L)(chunk_step)
    pltpu.sync_copy(stage_v, yimpl_sh.at[pl.ds(s * UPS * N_F, UPS * N_F)])

  plsc.subcore_barrier()
  pltpu.sync_copy(yimpl_sh, yimpl_v)
  for d in db:
    d.wait()

  def p2_step(i):
    rowv = iota + i * L
    u = plsc.load_gather(idx_v, [rowv])
    it = plsc.load_gather(idx_v, [rowv + BPW])
    bbv = plsc.load_gather(idx_v, [rowv + 2 * BPW])
    tdv = plsc.load_gather(idx_v, [rowv + 3 * BPW])
    mdv = plsc.load_gather(idx_v, [rowv + 4 * BPW])
    mean = plsc.load_gather(tab_v, [u + F_MU])
    d = tdv.astype(jnp.float32) - mean
    dev = jnp.sign(d) * jnp.exp(BETA * _ln(jnp.abs(d)))
    but = (plsc.load_gather(tab_v, [u + F_BU])
           + plsc.load_gather(tab_v, [u + F_AL]) * dev
           + plsc.load_gather(tab_v, [mdv + F_BTD]))
    cui = (plsc.load_gather(tab_v, [u + F_BCU])
           + plsc.load_gather(tab_v, [mdv + F_WCU]))
    bit = (plsc.load_gather(tab_v, [it + F_BI])
           + plsc.load_gather(tab_v, [it * BIN + bbv + F_WBIT])) * cui
    uf = u * N_F
    yf = (u + jnp.where(u >= USPLIT, 4, 0)) * N_F
    itf = it * N_F
    mdf = mdv * N_F
    bv = jnp.zeros((L,), jnp.float32)
    for f in range(N_F):
      uvec = (plsc.load_gather(tab_v, [uf + (F_WPU + f)])
              + plsc.load_gather(yimpl_v, [yf + f])
              + plsc.load_gather(tab_v, [uf + (F_AUK + f)]) * dev
              + plsc.load_gather(tab_v, [mdf + (F_PKUT + f)]))
      bv = bv + uvec * plsc.load_gather(tab_v, [itf + (F_WPI + f)])
    pred = GMEAN + but + bit + bv
    obase = rowv * N_F
    for f in range(N_F):
      wf = plsc.load_gather(tab_v, [jnp.full((L,), F_OW + f, jnp.int32)])
      bf = plsc.load_gather(tab_v, [jnp.full((L,), F_OB + f, jnp.int32)])
      plsc.store_scatter(out_v, [obase + f], pred * wf + bf)

  plsc.parallel_loop(0, NCHUNK, unroll=2)(p2_step)
  pltpu.sync_copy(out_v, out.at[pl.ds(base * N_F, BPW * N_F)])


@functools.lru_cache(maxsize=1)
def _build():
  mesh = plsc.VectorSubcoreMesh(
      core_axis_name="c", subcore_axis_name="s", num_cores=1, num_subcores=NS)
  return pl.kernel(
      _body,
      out_type=jax.ShapeDtypeStruct((B * N_F,), jnp.float32),
      mesh=mesh,
      compiler_params=pltpu.CompilerParams(needs_layout_passes=False),
      scratch_types=[
          pltpu.VMEM((F_TOTAL,), jnp.float32),
          pltpu.VMEM((UPS * HIST,), jnp.int32),
          pltpu.VMEM((N_USERS,), jnp.int32),
          pltpu.VMEM((5 * BPW,), jnp.int32),
          pltpu.VMEM((UPAD * N_F,), jnp.float32),
          pltpu.VMEM((UPS * N_F,), jnp.float32),
          pltpu.VMEM((BPW * N_F,), jnp.float32),
          pltpu.VMEM_SHARED((UPAD * N_F,), jnp.float32),
          pltpu.SemaphoreType.DMA,
          pltpu.SemaphoreType.DMA,
      ],
      name="rec_implicit_sc",
  )


def _pack_f32(parts):
  pieces = []
  off = 0
  for want, arr in parts:
    if want > off:
      pieces.append(jnp.zeros(want - off, jnp.float32))
      off = want
    pieces.append(arr)
    off += arr.shape[0]
  if F_TOTAL > off:
    pieces.append(jnp.zeros(F_TOTAL - off, jnp.float32))
  return jnp.concatenate(pieces)


def kernel(user_ids, item_ids, ITBin, tday, maxday_cat, mean_ud,
           user_itemcount, user_rated_item, BU, BI, WPU, WPI, WBIT, AlphaUK,
           WPUKT, Alpha, BTDay, BCU, WCU, Y, out_W, out_b):
  fblob = _pack_f32([
      (F_Y, Y.reshape(-1)), (F_BU, BU.reshape(-1)), (F_AL, Alpha.reshape(-1)),
      (F_MU, mean_ud.reshape(-1)), (F_BCU, BCU), (F_WPU, WPU.reshape(-1)),
      (F_AUK, AlphaUK.reshape(-1)), (F_BI, BI.reshape(-1)),
      (F_WPI, WPI.reshape(-1)), (F_WBIT, WBIT.reshape(-1)), (F_BTD, BTDay),
      (F_WCU, WCU.reshape(-1)), (F_PKUT, WPUKT.reshape(-1)),
      (F_OW, out_W.reshape(-1)), (F_OB, out_b)])
  idx = jnp.stack([user_ids, item_ids, ITBin, tday, maxday_cat]
                  ).astype(jnp.int32)
  idx_tm = idx.reshape(5, NS, BPW).transpose(1, 0, 2).reshape(-1)
  iblob = jnp.concatenate([
      user_rated_item.reshape(-1).astype(jnp.int32),
      jnp.zeros(I_CNT - N_USERS * HIST, jnp.int32),
      user_itemcount.astype(jnp.int32),
      jnp.zeros(I_IDX - I_CNT - N_USERS, jnp.int32),
      idx_tm])
  fn = _build()
  return fn(fblob, iblob).reshape(B, N_F)


# speedup vs baseline: 1.3649x; 1.3649x over previous
import functools

import jax
import jax.numpy as jnp
from jax import lax
from jax.experimental import pallas as pl
from jax.experimental.pallas import tpu as pltpu
from jax.experimental.pallas import tpu_sc as plsc

N_USERS = 1340
N_ITEMS = 733
N_F = 5
BIN = 60
MAXDAY = 4097
B = 16384
HIST = 50
BETA = 0.4
GMEAN = 4.0

NC = 1
NS = 16
NW = NC * NS
L = 16

BPW = B // NW
UPS = 96
NT_P1 = 14
LASTOFF = N_USERS - UPS
USPLIT = 13 * UPS
UPAD = NT_P1 * UPS
NCHUNK = BPW // L

LN2 = 0.6931471805599453
SQRT2 = 1.4142135623730951


def _ln(x):
  bits = lax.bitcast_convert_type(x, jnp.int32)
  e = lax.shift_right_logical(bits, 23) - 127
  m = lax.bitcast_convert_type(
      jnp.bitwise_or(jnp.bitwise_and(bits, 0x007FFFFF), 0x3F800000),
      jnp.float32)
  big = m > SQRT2
  m = jnp.where(big, m * 0.5, m)
  e = (e + jnp.where(big, 1, 0)).astype(jnp.float32)
  z = (m - 1.0) / (m + 1.0)
  z2 = z * z
  p = z * (2.0 + z2 * (2.0 / 3.0 + z2 * (0.4 + z2 * (2.0 / 7.0 + z2 * (2.0 / 9.0)))))
  return e * LN2 + p


def _body(uri, cnt, y, bu, al, mu, bcu, wpu, auk, bi, wpi, wbit, btd, wcu,
          pkut, ow, ob, uid, iid, tbin, td, md, out,
          uri_v, cnt_v, y_v, bu_v, al_v, mu_v, bcu_v, wpu_v, auk_v,
          bi_v, wpi_v, wbit_v, btd_v, wcu_v, pkut_v, w_v, b_v,
          u_v, it_v, bb_v, td_v, md_v, yimpl_v, stage_v, out_v, yimpl_sh,
          sem_a, sem_b):
  c = lax.axis_index("c")
  s = lax.axis_index("s")
  wid = c * NS + s
  base = wid * BPW
  iota = lax.iota(jnp.int32, L)

  off_u = jnp.minimum(s * UPS, LASTOFF)
  da = [pltpu.async_copy(uri.at[pl.ds(off_u * HIST, UPS * HIST)], uri_v, sem_a),
        pltpu.async_copy(cnt, cnt_v, sem_a),
        pltpu.async_copy(y, y_v, sem_a)]
  db = [pltpu.async_copy(src, dst, sem_b) for src, dst in
        ((bu, bu_v), (al, al_v), (mu, mu_v), (bcu, bcu_v), (wpu, wpu_v),
         (auk, auk_v), (bi, bi_v), (wpi, wpi_v), (wbit, wbit_v),
         (btd, btd_v), (wcu, wcu_v), (pkut, pkut_v))]
  db += [pltpu.async_copy(ow, w_v.at[pl.ds(8, N_F)], sem_b),
         pltpu.async_copy(ob, b_v.at[pl.ds(8, N_F)], sem_b)]
  db += [pltpu.async_copy(src.at[pl.ds(base, BPW)], dst, sem_b) for src, dst in
         ((uid, u_v), (iid, it_v), (tbin, bb_v), (td, td_v), (md, md_v))]
  for d in da:
    d.wait()

  @pl.when(s < NT_P1)
  def _phase1():
    for chunk in range(UPS // L):
      rows = iota + chunk * L
      rbase = rows * HIST

      def h_step(h, accs):
        hidx = plsc.load_gather(uri_v, [rbase + h])
        ybase = hidx * N_F
        return tuple(
            acc + plsc.load_gather(y_v, [ybase + f])
            for f, acc in enumerate(accs))

      accs = plsc.parallel_loop(
          0, HIST, unroll=2,
          carry=tuple(jnp.zeros((L,), jnp.float32) for _ in range(N_F)))(h_step)
      cntf = plsc.load_gather(cnt_v, [off_u + rows]).astype(jnp.float32)
      ru = jnp.exp(-0.5 * _ln(cntf))
      sbase = rows * N_F
      for f in range(N_F):
        plsc.store_scatter(stage_v, [sbase + f], accs[f] * ru)

    pltpu.sync_copy(stage_v, yimpl_sh.at[pl.ds(s * UPS * N_F, UPS * N_F)])

  plsc.subcore_barrier()
  pltpu.sync_copy(yimpl_sh, yimpl_v)
  for d in db:
    d.wait()

  wvals = [plsc.load_gather(w_v, [jnp.full((L,), 8 + f, jnp.int32)])
           for f in range(N_F)]
  bvals = [plsc.load_gather(b_v, [jnp.full((L,), 8 + f, jnp.int32)])
           for f in range(N_F)]

  def p2_step(i, carry):
    off = i * L
    rowv = iota + off
    u = u_v[pl.ds(off, L)]
    it = it_v[pl.ds(off, L)]
    bbv = bb_v[pl.ds(off, L)]
    tdv = td_v[pl.ds(off, L)]
    mdv = md_v[pl.ds(off, L)]
    mean = plsc.load_gather(mu_v, [u])
    d = tdv.astype(jnp.float32) - mean
    dev = jnp.sign(d) * jnp.exp(BETA * _ln(jnp.abs(d)))
    but = (plsc.load_gather(bu_v, [u]) + plsc.load_gather(al_v, [u]) * dev
           + plsc.load_gather(btd_v, [mdv]))
    cui = plsc.load_gather(bcu_v, [u]) + plsc.load_gather(wcu_v, [mdv])
    bit = (plsc.load_gather(bi_v, [it])
           + plsc.load_gather(wbit_v, [it * BIN + bbv])) * cui
    uf = u * N_F
    yf = (u + jnp.where(u >= USPLIT, 4, 0)) * N_F
    itf = it * N_F
    mdf = mdv * N_F
    bv = jnp.zeros((L,), jnp.float32)
    for f in range(N_F):
      uvec = (plsc.load_gather(wpu_v, [uf + f])
              + plsc.load_gather(yimpl_v, [yf + f])
              + plsc.load_gather(auk_v, [uf + f]) * dev
              + plsc.load_gather(pkut_v, [mdf + f]))
      bv = bv + uvec * plsc.load_gather(wpi_v, [itf + f])
    pred = GMEAN + but + bit + bv
    obase = rowv * N_F
    for f in range(N_F):
      plsc.store_scatter(out_v, [obase + f], pred * wvals[f] + bvals[f])

  plsc.parallel_loop(0, NCHUNK, unroll=2)(
      lambda i: p2_step(i, None))
  pltpu.sync_copy(out_v, out.at[pl.ds(base * N_F, BPW * N_F)])


@functools.lru_cache(maxsize=1)
def _build():
  mesh = plsc.VectorSubcoreMesh(
      core_axis_name="c", subcore_axis_name="s", num_cores=NC, num_subcores=NS)
  return pl.kernel(
      _body,
      out_type=jax.ShapeDtypeStruct((B * N_F,), jnp.float32),
      mesh=mesh,
      compiler_params=pltpu.CompilerParams(needs_layout_passes=False),
      scratch_types=[
          pltpu.VMEM((UPS * HIST,), jnp.int32),
          pltpu.VMEM((N_USERS,), jnp.int32),
          pltpu.VMEM((N_USERS * N_F,), jnp.float32),
          pltpu.VMEM((N_USERS,), jnp.float32),
          pltpu.VMEM((N_USERS,), jnp.float32),
          pltpu.VMEM((N_USERS,), jnp.float32),
          pltpu.VMEM((N_USERS,), jnp.float32),
          pltpu.VMEM((N_USERS * N_F,), jnp.float32),
          pltpu.VMEM((N_USERS * N_F,), jnp.float32),
          pltpu.VMEM((N_ITEMS,), jnp.float32),
          pltpu.VMEM((N_ITEMS * N_F,), jnp.float32),
          pltpu.VMEM((N_ITEMS * BIN,), jnp.float32),
          pltpu.VMEM((MAXDAY,), jnp.float32),
          pltpu.VMEM((MAXDAY,), jnp.float32),
          pltpu.VMEM((MAXDAY * N_F,), jnp.float32),
          pltpu.VMEM((L,), jnp.float32),
          pltpu.VMEM((L,), jnp.float32),
          pltpu.VMEM((BPW,), jnp.int32),
          pltpu.VMEM((BPW,), jnp.int32),
          pltpu.VMEM((BPW,), jnp.int32),
          pltpu.VMEM((BPW,), jnp.int32),
          pltpu.VMEM((BPW,), jnp.int32),
          pltpu.VMEM((UPAD * N_F,), jnp.float32),
          pltpu.VMEM((UPS * N_F,), jnp.float32),
          pltpu.VMEM((BPW * N_F,), jnp.float32),
          pltpu.VMEM_SHARED((UPAD * N_F,), jnp.float32),
          pltpu.SemaphoreType.DMA,
          pltpu.SemaphoreType.DMA,
      ],
      name="rec_implicit_sc",
  )


def kernel(user_ids, item_ids, ITBin, tday, maxday_cat, mean_ud,
           user_itemcount, user_rated_item, BU, BI, WPU, WPI, WBIT, AlphaUK,
           WPUKT, Alpha, BTDay, BCU, WCU, Y, out_W, out_b):
  fn = _build()
  flat = fn(user_rated_item.reshape(-1).astype(jnp.int32),
            user_itemcount.astype(jnp.int32), Y.reshape(-1),
            BU.reshape(-1), Alpha.reshape(-1), mean_ud.reshape(-1), BCU,
            WPU.reshape(-1), AlphaUK.reshape(-1),
            BI.reshape(-1), WPI.reshape(-1), WBIT.reshape(-1), BTDay,
            WCU.reshape(-1), WPUKT.reshape(-1), out_W.reshape(-1), out_b,
            user_ids.astype(jnp.int32), item_ids.astype(jnp.int32),
            ITBin.astype(jnp.int32), tday.astype(jnp.int32),
            maxday_cat.astype(jnp.int32))
  return flat.reshape(B, N_F)
